# baseline (speedup 1.0000x reference)
import functools
import math

import jax
import jax.numpy as jnp
from jax import lax
from jax.experimental import pallas as pl
from jax.experimental.pallas import tpu as pltpu
from jax.experimental.pallas import tpu_sc as plsc

H = 8
P = 6
NW = 32
K = 4 * H * P


def _vproj_body(v_ref, w_ref, b_ref, o_ref):
    o_ref[...] = jnp.dot(v_ref[...], w_ref[...],
                         preferred_element_type=jnp.float32) + b_ref[...]


def _vproj(value2d, W_v, b_v, blk=1024):
    n, d = value2d.shape
    return pl.pallas_call(
        _vproj_body,
        grid=(n // blk,),
        in_specs=[
            pl.BlockSpec((blk, d), lambda i: (i, 0)),
            pl.BlockSpec((d, d), lambda i: (0, 0)),
            pl.BlockSpec((1, d), lambda i: (0, 0)),
        ],
        out_specs=pl.BlockSpec((blk, d), lambda i: (i, 0)),
        out_shape=jax.ShapeDtypeStruct((n, d), jnp.float32),
    )(value2d, W_v, b_v.reshape(1, d))


def _sample_body(nq_blocks_per_batch, hw_side, q_ref, rp_ref, wx_ref, bx_ref,
                 wy_ref, by_ref, wa_ref, ba_ref, idx_ref, coef_ref):
    b = pl.program_id(0) // nq_blocks_per_batch
    q = q_ref[...]
    bq = q.shape[0]
    hp = H * P

    ox = jnp.dot(q, wx_ref[...], preferred_element_type=jnp.float32) + bx_ref[...]
    oy = jnp.dot(q, wy_ref[...], preferred_element_type=jnp.float32) + by_ref[...]
    al = jnp.dot(q, wa_ref[...], preferred_element_type=jnp.float32) + ba_ref[...]

    a3 = al.reshape(bq, H, P)
    a3 = a3 - jnp.max(a3, axis=-1, keepdims=True)
    e = jnp.exp(a3)
    attn = (e / jnp.sum(e, axis=-1, keepdims=True)).reshape(bq, hp)

    wf = jnp.float32(hw_side)
    x = (rp_ref[:, 0:1] + ox * (1.0 / wf)) * wf - 0.5
    y = (rp_ref[:, 1:2] + oy * (1.0 / wf)) * wf - 0.5
    x0 = jnp.floor(x)
    y0 = jnp.floor(y)
    fx = x - x0
    fy = y - y0

    xb = jnp.clip(x0, 0.0, wf - 2.0)
    yb = jnp.clip(y0, 0.0, wf - 2.0)

    h_lane = jax.lax.broadcasted_iota(jnp.int32, (bq, hp), 1) // P
    base = b * (hw_side * hw_side * H) + h_lane
    spat = yb.astype(jnp.int32) * hw_side + xb.astype(jnp.int32)
    idx_ref[...] = base + spat * H

    for s in range(4):
        sy, sx = float(s // 2), float(s % 2)
        coef_s = jnp.zeros((bq, hp), jnp.float32)
        for dy, dx in ((0, 0), (0, 1), (1, 0), (1, 1)):
            xi = x0 + dx
            yi = y0 + dy
            valid = ((xi >= 0) & (xi < wf) & (yi >= 0) & (yi < wf))
            wgt = (fx if dx else 1.0 - fx) * (fy if dy else 1.0 - fy)
            hit = valid & (yi - yb == sy) & (xi - xb == sx)
            coef_s = coef_s + jnp.where(hit, wgt * attn, 0.0)
        coef_ref[:, s, :] = coef_s


def _sample_params(q2d, rp2d, Wx, bx, Wy, by, Wa, ba, nq, hw_side, blk=1024):
    n, d = q2d.shape
    hp = H * P
    body = functools.partial(_sample_body, nq // blk, hw_side)
    return pl.pallas_call(
        body,
        grid=(n // blk,),
        in_specs=[
            pl.BlockSpec((blk, d), lambda i: (i, 0)),
            pl.BlockSpec((blk, 2), lambda i: (i, 0)),
            pl.BlockSpec((d, hp), lambda i: (0, 0)),
            pl.BlockSpec((1, hp), lambda i: (0, 0)),
            pl.BlockSpec((d, hp), lambda i: (0, 0)),
            pl.BlockSpec((1, hp), lambda i: (0, 0)),
            pl.BlockSpec((d, hp), lambda i: (0, 0)),
            pl.BlockSpec((1, hp), lambda i: (0, 0)),
        ],
        out_specs=[
            pl.BlockSpec((blk, hp), lambda i: (i, 0)),
            pl.BlockSpec((blk, 4, hp), lambda i: (i, 0, 0)),
        ],
        out_shape=[
            jax.ShapeDtypeStruct((n, hp), jnp.int32),
            jax.ShapeDtypeStruct((n, 4, hp), jnp.float32),
        ],
    )(q2d, rp2d, Wx, bx.reshape(1, hp), Wy, by.reshape(1, hp),
      Wa, ba.reshape(1, hp))


HP = H * P
CHUNKQ = 16
KPC = CHUNKQ * HP
NSTREAM = KPC // 128
SLAB = 8
CPH = 32
CPC = CHUNKQ * H * CPH


def _gather_kernel(nrows, d_model):
    rows_per_w = nrows // NW
    n_chunks = rows_per_w // CHUNKQ
    mesh = plsc.VectorSubcoreMesh(core_axis_name="c", subcore_axis_name="s")

    @functools.partial(
        pl.kernel,
        mesh=mesh,
        out_type=jax.ShapeDtypeStruct((nrows, d_model), jnp.float32),
        scratch_types=[
            pltpu.VMEM((SLAB, 128), jnp.int32),
            pltpu.VMEM((CPC,), jnp.float32),
            pltpu.VMEM((KPC, 128), jnp.float32),
            pltpu.VMEM((CHUNKQ, d_model), jnp.float32),
            pltpu.SemaphoreType.DMA,
        ],
    )
    def body(table, idx_p, coef1d, out, idx_v, coef_v, rows_v, out_v, sem):
        wid = lax.axis_index("s") * 2 + lax.axis_index("c")
        wrow0 = wid * rows_per_w
        wch0 = wid * n_chunks

        def chunk(ch, _):
            gch = wch0 + ch
            pltpu.sync_copy(idx_p.at[pl.ds(gch * SLAB, SLAB)], idx_v)
            pltpu.sync_copy(coef1d.at[pl.ds(gch * CPC, CPC)], coef_v)
            copies = [
                pltpu.async_copy(table.at[idx_v.at[j]],
                                 rows_v.at[pl.ds(j * 128, 128)], sem)
                for j in range(NSTREAM)
            ]
            for cp in copies:
                cp.wait()

            @plsc.parallel_loop(0, CHUNKQ)
            def one_row(r):
                cb = r * (H * CPH)
                rb = r * HP
                for h in range(H):
                    ca = coef_v[pl.ds(cb + CPH * h, 16)]
                    cc = coef_v[pl.ds(cb + CPH * h + 16, 16)]
                    acc0 = jnp.zeros((16,), jnp.float32)
                    acc1 = jnp.zeros((16,), jnp.float32)
                    for p in range(P):
                        e = rb + P * h + p
                        for s in range(4):
                            t = 4 * p + s
                            cs = ca[t] if t < 16 else cc[t - 16]
                            v0 = rows_v[e, pl.ds(32 * s, 16)]
                            v1 = rows_v[e, pl.ds(32 * s + 16, 16)]
                            acc0 = acc0 + cs * v0
                            acc1 = acc1 + cs * v1
                    out_v[r, pl.ds(32 * h, 16)] = acc0
                    out_v[r, pl.ds(32 * h + 16, 16)] = acc1

            pltpu.sync_copy(out_v, out.at[pl.ds(wrow0 + ch * CHUNKQ, CHUNKQ)])
            return 0

        lax.fori_loop(0, n_chunks, chunk, 0)

    return body


def _ln(x, g, b):
    mu = jnp.mean(x, axis=-1, keepdims=True)
    xc = x - mu
    var = jnp.mean(xc * xc, axis=-1, keepdims=True)
    return xc * jax.lax.rsqrt(var + 1e-5) * g + b


def _mlp_body(s_ref, q_ref, wo_ref, bo_ref, g1_ref, c1_ref, w1_ref, b1_ref,
              w2_ref, b2_ref, g2_ref, c2_ref, o_ref):
    s = jnp.dot(s_ref[...], wo_ref[...], preferred_element_type=jnp.float32)
    s = s + bo_ref[...] + q_ref[...]
    y1 = _ln(s, g1_ref[...], c1_ref[...])
    t = jnp.dot(y1, w1_ref[...], preferred_element_type=jnp.float32) + b1_ref[...]
    gelu = 0.5 * t * (1.0 + jax.lax.erf(t * (1.0 / math.sqrt(2.0))))
    y2 = jnp.dot(gelu, w2_ref[...], preferred_element_type=jnp.float32) + b2_ref[...]
    o_ref[...] = _ln(y2 + y1, g2_ref[...], c2_ref[...])


def _out_mlp(sampled, q2d, W_out, b_out, ln1_g, ln1_b, W1, b1, W2, b2,
             ln2_g, ln2_b, blk=1024):
    n, d = q2d.shape
    row = lambda a: a.reshape(1, d)
    mat_spec = pl.BlockSpec((d, d), lambda i: (0, 0))
    vec_spec = pl.BlockSpec((1, d), lambda i: (0, 0))
    return pl.pallas_call(
        _mlp_body,
        grid=(n // blk,),
        in_specs=[
            pl.BlockSpec((blk, d), lambda i: (i, 0)),
            pl.BlockSpec((blk, d), lambda i: (i, 0)),
            mat_spec, vec_spec, vec_spec, vec_spec,
            mat_spec, vec_spec, mat_spec, vec_spec,
            vec_spec, vec_spec,
        ],
        out_specs=pl.BlockSpec((blk, d), lambda i: (i, 0)),
        out_shape=jax.ShapeDtypeStruct((n, d), jnp.float32),
    )(sampled, q2d, W_out, row(b_out), row(ln1_g), row(ln1_b),
      W1, row(b1), W2, row(b2), row(ln2_g), row(ln2_b))


def kernel(query, value, reference_points, W_off, b_off, W_attn, b_attn,
           W_v, b_v, W_out, b_out, ln1_g, ln1_b, W1, b1, W2, b2,
           ln2_g, ln2_b, spatial_shapes, level_start_index):
    B, Nq, d = query.shape
    HW = value.shape[1]
    side = math.isqrt(HW)
    c = d // H

    vproj = _vproj(value.reshape(B * HW, d), W_v, b_v)
    v4 = vproj.reshape(B, side, side, d)
    tx = jnp.concatenate([v4[:, :, 1:], v4[:, :, -1:]], axis=2)
    ty = jnp.concatenate([v4[:, 1:], v4[:, -1:]], axis=1)
    txy = jnp.concatenate([ty[:, :, 1:], ty[:, :, -1:]], axis=2)
    corners = [a.reshape(B, side, side, H, 1, c) for a in (v4, tx, ty, txy)]
    table = jnp.concatenate(corners, axis=4).reshape(B * HW * H, 4 * c)

    Wo3 = W_off.reshape(d, H * P, 2)
    bo2 = b_off.reshape(H * P, 2)
    q2d = query.reshape(B * Nq, d)
    rp2d = reference_points.reshape(B * Nq, 2)
    idx, coef = _sample_params(q2d, rp2d, Wo3[:, :, 0], bo2[:, 0],
                               Wo3[:, :, 1], bo2[:, 1], W_attn, b_attn,
                               Nq, side)

    nchunk = B * Nq // CHUNKQ
    idx_p = jnp.pad(idx.reshape(nchunk, KPC), ((0, 0), (0, SLAB * 128 - KPC)))
    idx_p = idx_p.reshape(nchunk * SLAB, 128)
    coef_h = jnp.pad(coef.reshape(-1, 4, H, P).transpose(0, 2, 3, 1)
                     .reshape(-1, H, P * 4), ((0, 0), (0, 0), (0, CPH - P * 4)))
    gather = _gather_kernel(B * Nq, d)
    sampled = gather(table, idx_p, coef_h.reshape(-1))

    out = _out_mlp(sampled, q2d, W_out, b_out, ln1_g, ln1_b,
                   W1, b1, W2, b2, ln2_g, ln2_b)
    return out.reshape(B, Nq, d)


# speedup vs baseline: 1.2129x; 1.2129x over previous
import functools
import math

import jax
import jax.numpy as jnp
from jax import lax
from jax.experimental import pallas as pl
from jax.experimental.pallas import tpu as pltpu
from jax.experimental.pallas import tpu_sc as plsc

H = 8
P = 6
NW = 32
K = 4 * H * P


def _vproj_body(v_ref, w_ref, b_ref, o_ref):
    o_ref[...] = jnp.dot(v_ref[...], w_ref[...],
                         preferred_element_type=jnp.float32) + b_ref[...]


def _vproj(value2d, W_v, b_v, blk=1024):
    n, d = value2d.shape
    return pl.pallas_call(
        _vproj_body,
        grid=(n // blk,),
        in_specs=[
            pl.BlockSpec((blk, d), lambda i: (i, 0)),
            pl.BlockSpec((d, d), lambda i: (0, 0)),
            pl.BlockSpec((1, d), lambda i: (0, 0)),
        ],
        out_specs=pl.BlockSpec((blk, d), lambda i: (i, 0)),
        out_shape=jax.ShapeDtypeStruct((n, d), jnp.float32),
    )(value2d, W_v, b_v.reshape(1, d))


def _sample_body(nq_blocks_per_batch, hw_side, q_ref, rp_ref, wx_ref, bx_ref,
                 wy_ref, by_ref, wa_ref, ba_ref, idx_ref, coef_ref):
    b = pl.program_id(0) // nq_blocks_per_batch
    q = q_ref[...]
    bq = q.shape[0]
    hp = H * P

    ox = jnp.dot(q, wx_ref[...], preferred_element_type=jnp.float32) + bx_ref[...]
    oy = jnp.dot(q, wy_ref[...], preferred_element_type=jnp.float32) + by_ref[...]
    al = jnp.dot(q, wa_ref[...], preferred_element_type=jnp.float32) + ba_ref[...]

    a3 = al.reshape(bq, H, P)
    a3 = a3 - jnp.max(a3, axis=-1, keepdims=True)
    e = jnp.exp(a3)
    attn = (e / jnp.sum(e, axis=-1, keepdims=True)).reshape(bq, hp)

    wf = jnp.float32(hw_side)
    x = (rp_ref[:, 0:1] + ox * (1.0 / wf)) * wf - 0.5
    y = (rp_ref[:, 1:2] + oy * (1.0 / wf)) * wf - 0.5
    x0 = jnp.floor(x)
    y0 = jnp.floor(y)
    fx = x - x0
    fy = y - y0

    xb = jnp.clip(x0, 0.0, wf - 2.0)
    yb = jnp.clip(y0, 0.0, wf - 2.0)

    h_lane = jax.lax.broadcasted_iota(jnp.int32, (bq, hp), 1) // P
    base = b * (hw_side * hw_side * H) + h_lane
    spat = yb.astype(jnp.int32) * hw_side + xb.astype(jnp.int32)
    idx_ref[...] = base + spat * H

    for s in range(4):
        sy, sx = float(s // 2), float(s % 2)
        coef_s = jnp.zeros((bq, hp), jnp.float32)
        for dy, dx in ((0, 0), (0, 1), (1, 0), (1, 1)):
            xi = x0 + dx
            yi = y0 + dy
            valid = ((xi >= 0) & (xi < wf) & (yi >= 0) & (yi < wf))
            wgt = (fx if dx else 1.0 - fx) * (fy if dy else 1.0 - fy)
            hit = valid & (yi - yb == sy) & (xi - xb == sx)
            coef_s = coef_s + jnp.where(hit, wgt * attn, 0.0)
        coef_ref[:, s, :] = coef_s


def _sample_params(q2d, rp2d, Wx, bx, Wy, by, Wa, ba, nq, hw_side, blk=1024):
    n, d = q2d.shape
    hp = H * P
    body = functools.partial(_sample_body, nq // blk, hw_side)
    return pl.pallas_call(
        body,
        grid=(n // blk,),
        in_specs=[
            pl.BlockSpec((blk, d), lambda i: (i, 0)),
            pl.BlockSpec((blk, 2), lambda i: (i, 0)),
            pl.BlockSpec((d, hp), lambda i: (0, 0)),
            pl.BlockSpec((1, hp), lambda i: (0, 0)),
            pl.BlockSpec((d, hp), lambda i: (0, 0)),
            pl.BlockSpec((1, hp), lambda i: (0, 0)),
            pl.BlockSpec((d, hp), lambda i: (0, 0)),
            pl.BlockSpec((1, hp), lambda i: (0, 0)),
        ],
        out_specs=[
            pl.BlockSpec((blk, hp), lambda i: (i, 0)),
            pl.BlockSpec((blk, 4, hp), lambda i: (i, 0, 0)),
        ],
        out_shape=[
            jax.ShapeDtypeStruct((n, hp), jnp.int32),
            jax.ShapeDtypeStruct((n, 4, hp), jnp.float32),
        ],
    )(q2d, rp2d, Wx, bx.reshape(1, hp), Wy, by.reshape(1, hp),
      Wa, ba.reshape(1, hp))


HP = H * P
CHUNKQ = 8
KPC = CHUNKQ * HP
NSTREAM = KPC // 128
SLAB = 8
CPC = CHUNKQ * 4 * HP


def _gather_kernel(nrows, d_model):
    rows_per_w = nrows // NW
    n_chunks = rows_per_w // CHUNKQ
    mesh = plsc.VectorSubcoreMesh(core_axis_name="c", subcore_axis_name="s")

    @functools.partial(
        pl.kernel,
        mesh=mesh,
        out_type=jax.ShapeDtypeStruct((nrows, d_model), jnp.float32),
        scratch_types=[
            pltpu.VMEM((2, SLAB, 128), jnp.int32),
            pltpu.VMEM((2, CPC), jnp.float32),
            pltpu.VMEM((2, KPC, 128), jnp.float32),
            pltpu.VMEM((CHUNKQ, d_model), jnp.float32),
            pltpu.SemaphoreType.DMA,
            pltpu.SemaphoreType.DMA,
        ],
    )
    def body(table, idx_p, coef1d, out, idx_v, coef_v, rows_v, out_v, s0, s1):
        wid = lax.axis_index("s") * 2 + lax.axis_index("c")
        wrow0 = wid * rows_per_w
        wch0 = wid * n_chunks
        sems = (s0, s1)

        def stage(ci, b):
            gch = wch0 + ci
            pltpu.sync_copy(idx_p.at[pl.ds(gch * SLAB, SLAB)], idx_v.at[b])
            pltpu.sync_copy(coef1d.at[pl.ds(gch * CPC, CPC)], coef_v.at[b])
            for j in range(NSTREAM):
                pltpu.async_copy(table.at[idx_v.at[b].at[j]],
                                 rows_v.at[b].at[pl.ds(j * 128, 128)], sems[b])

        def compute(ch, b):
            for j in range(NSTREAM):
                pltpu.make_async_copy(
                    table.at[idx_v.at[b].at[j]],
                    rows_v.at[b].at[pl.ds(j * 128, 128)], sems[b]).wait()

            @plsc.parallel_loop(0, CHUNKQ)
            def one_row(r):
                cb = r * (4 * HP)
                rb = r * HP
                acc = [jnp.zeros((16,), jnp.float32)
                       for _ in range(d_model // 16)]
                for g in range(HP // 16):
                    cg = [coef_v[b, pl.ds(cb + HP * s + 16 * g, 16)]
                          for s in range(4)]
                    for j in range(16):
                        e = 16 * g + j
                        h = e // P
                        for s in range(4):
                            cs = cg[s][j]
                            v0 = rows_v[b, rb + e, pl.ds(32 * s, 16)]
                            v1 = rows_v[b, rb + e, pl.ds(32 * s + 16, 16)]
                            acc[2 * h] = acc[2 * h] + cs * v0
                            acc[2 * h + 1] = acc[2 * h + 1] + cs * v1
                for i in range(d_model // 16):
                    out_v[r, pl.ds(16 * i, 16)] = acc[i]

            pltpu.sync_copy(out_v, out.at[pl.ds(wrow0 + ch * CHUNKQ, CHUNKQ)])

        stage(0, 0)

        def pair(it, _):
            ch = it * 2
            stage(jnp.minimum(ch + 1, n_chunks - 1), 1)
            compute(ch, 0)
            stage(jnp.minimum(ch + 2, n_chunks - 1), 0)
            compute(ch + 1, 1)
            return 0

        lax.fori_loop(0, n_chunks // 2, pair, 0)
        for j in range(NSTREAM):
            pltpu.make_async_copy(
                table.at[idx_v.at[0].at[j]],
                rows_v.at[0].at[pl.ds(j * 128, 128)], sems[0]).wait()

    return body


def _ln(x, g, b):
    mu = jnp.mean(x, axis=-1, keepdims=True)
    xc = x - mu
    var = jnp.mean(xc * xc, axis=-1, keepdims=True)
    return xc * jax.lax.rsqrt(var + 1e-5) * g + b


def _mlp_body(s_ref, q_ref, wo_ref, bo_ref, g1_ref, c1_ref, w1_ref, b1_ref,
              w2_ref, b2_ref, g2_ref, c2_ref, o_ref):
    s = jnp.dot(s_ref[...], wo_ref[...], preferred_element_type=jnp.float32)
    s = s + bo_ref[...] + q_ref[...]
    y1 = _ln(s, g1_ref[...], c1_ref[...])
    t = jnp.dot(y1, w1_ref[...], preferred_element_type=jnp.float32) + b1_ref[...]
    gelu = 0.5 * t * (1.0 + jax.lax.erf(t * (1.0 / math.sqrt(2.0))))
    y2 = jnp.dot(gelu, w2_ref[...], preferred_element_type=jnp.float32) + b2_ref[...]
    o_ref[...] = _ln(y2 + y1, g2_ref[...], c2_ref[...])


def _out_mlp(sampled, q2d, W_out, b_out, ln1_g, ln1_b, W1, b1, W2, b2,
             ln2_g, ln2_b, blk=1024):
    n, d = q2d.shape
    row = lambda a: a.reshape(1, d)
    mat_spec = pl.BlockSpec((d, d), lambda i: (0, 0))
    vec_spec = pl.BlockSpec((1, d), lambda i: (0, 0))
    return pl.pallas_call(
        _mlp_body,
        grid=(n // blk,),
        in_specs=[
            pl.BlockSpec((blk, d), lambda i: (i, 0)),
            pl.BlockSpec((blk, d), lambda i: (i, 0)),
            mat_spec, vec_spec, vec_spec, vec_spec,
            mat_spec, vec_spec, mat_spec, vec_spec,
            vec_spec, vec_spec,
        ],
        out_specs=pl.BlockSpec((blk, d), lambda i: (i, 0)),
        out_shape=jax.ShapeDtypeStruct((n, d), jnp.float32),
    )(sampled, q2d, W_out, row(b_out), row(ln1_g), row(ln1_b),
      W1, row(b1), W2, row(b2), row(ln2_g), row(ln2_b))


def kernel(query, value, reference_points, W_off, b_off, W_attn, b_attn,
           W_v, b_v, W_out, b_out, ln1_g, ln1_b, W1, b1, W2, b2,
           ln2_g, ln2_b, spatial_shapes, level_start_index):
    B, Nq, d = query.shape
    HW = value.shape[1]
    side = math.isqrt(HW)
    c = d // H

    vproj = _vproj(value.reshape(B * HW, d), W_v, b_v)
    v4 = vproj.reshape(B, side, side, d)
    tx = jnp.concatenate([v4[:, :, 1:], v4[:, :, -1:]], axis=2)
    ty = jnp.concatenate([v4[:, 1:], v4[:, -1:]], axis=1)
    txy = jnp.concatenate([ty[:, :, 1:], ty[:, :, -1:]], axis=2)
    corners = [a.reshape(B, side, side, H, 1, c) for a in (v4, tx, ty, txy)]
    table = jnp.concatenate(corners, axis=4).reshape(B * HW * H, 4 * c)

    Wo3 = W_off.reshape(d, H * P, 2)
    bo2 = b_off.reshape(H * P, 2)
    q2d = query.reshape(B * Nq, d)
    rp2d = reference_points.reshape(B * Nq, 2)
    idx, coef = _sample_params(q2d, rp2d, Wo3[:, :, 0], bo2[:, 0],
                               Wo3[:, :, 1], bo2[:, 1], W_attn, b_attn,
                               Nq, side)

    nchunk = B * Nq // CHUNKQ
    idx_p = jnp.pad(idx.reshape(nchunk, KPC), ((0, 0), (0, SLAB * 128 - KPC)))
    idx_p = idx_p.reshape(nchunk * SLAB, 128)
    gather = _gather_kernel(B * Nq, d)
    sampled = gather(table, idx_p, coef.reshape(-1))

    out = _out_mlp(sampled, q2d, W_out, b_out, ln1_g, ln1_b,
                   W1, b1, W2, b2, ln2_g, ln2_b)
    return out.reshape(B, Nq, d)
